# baseline (speedup 1.0000x reference)
import jax
import jax.numpy as jnp
from jax.experimental import pallas as pl
from jax.experimental.pallas import tpu as pltpu

B, J, N = 16, 8, 96 * 64 * 64
C = N // 8


def _body(alpha_ref, maskT_ref, x_ref, o_ref):
    b = pl.program_id(0)
    logits = maskT_ref[:] * alpha_ref[0, 0]
    m = jnp.max(logits, axis=0, keepdims=True)
    e = jnp.exp(logits - m)
    wT = e / jnp.sum(e, axis=0, keepdims=True)
    lane = jax.lax.broadcasted_iota(jnp.int32, (J, B), 1)
    wcol = jnp.sum(jnp.where(lane == b, wT, 0.0), axis=1, keepdims=True)
    o_ref[0, 0, :] = jnp.sum(x_ref[0] * wcol, axis=0)


def kernel(inps, mask, alpha):
    x = inps.reshape(B, J, N)
    maskT = mask.T
    alpha2 = jnp.reshape(alpha, (1, 1))
    out = pl.pallas_call(
        _body,
        grid=(B, N // C),
        in_specs=[
            pl.BlockSpec(memory_space=pltpu.SMEM),
            pl.BlockSpec((J, B), lambda b, c: (0, 0)),
            pl.BlockSpec((1, J, C), lambda b, c: (b, 0, c)),
        ],
        out_specs=pl.BlockSpec((1, 1, C), lambda b, c: (b * (N // C) + c, 0, 0)),
        out_shape=jax.ShapeDtypeStruct((B * (N // C), 1, C), jnp.float32),
        compiler_params=pltpu.CompilerParams(
            dimension_semantics=("parallel", "arbitrary"),
        ),
    )(alpha2, maskT, x)
    sampled = out.reshape(B, 96, 64, 64)
    logp = jnp.zeros((B,), jnp.float32)
    return (sampled, logp)


# speedup vs baseline: 1.9835x; 1.9835x over previous
import jax
import jax.numpy as jnp
from jax.experimental import pallas as pl
from jax.experimental.pallas import tpu as pltpu

B, J = 16, 8
R, L = 96, 64 * 64
SR = 8


def _body(alpha_ref, maskT_ref, x_ref, o_ref):
    b = pl.program_id(0)
    logits = maskT_ref[:] * alpha_ref[0, 0]
    m = jnp.max(logits, axis=0, keepdims=True)
    e = jnp.exp(logits - m)
    wT = e / jnp.sum(e, axis=0, keepdims=True)
    lane = jax.lax.broadcasted_iota(jnp.int32, (J, B), 1)
    wcol = jnp.sum(jnp.where(lane == b, wT, 0.0), axis=1)
    x = x_ref[0]
    acc = x[0] * wcol[0]
    for j in range(1, J):
        acc += x[j] * wcol[j]
    o_ref[0] = acc


def kernel(inps, mask, alpha):
    x = inps.reshape(B, J, R, L)
    maskT = mask.T
    alpha2 = jnp.reshape(alpha, (1, 1))
    out = pl.pallas_call(
        _body,
        grid=(B, R // SR),
        in_specs=[
            pl.BlockSpec(memory_space=pltpu.SMEM),
            pl.BlockSpec((J, B), lambda b, r: (0, 0)),
            pl.BlockSpec((1, J, SR, L), lambda b, r: (b, 0, r, 0)),
        ],
        out_specs=pl.BlockSpec((1, SR, L), lambda b, r: (b, r, 0)),
        out_shape=jax.ShapeDtypeStruct((B, R, L), jnp.float32),
        compiler_params=pltpu.CompilerParams(
            dimension_semantics=("parallel", "arbitrary"),
        ),
    )(alpha2, maskT, x)
    sampled = out.reshape(B, 96, 64, 64)
    logp = jnp.zeros((B,), jnp.float32)
    return (sampled, logp)


# speedup vs baseline: 2.3317x; 1.1756x over previous
import jax
import jax.numpy as jnp
from jax.experimental import pallas as pl
from jax.experimental.pallas import tpu as pltpu

B, J = 16, 8
R, L = 96, 64 * 64
SR = 8
RC = R // SR
NSTEP = B * RC
NBUF = 4


def _issue(x_hbm, buf, sems, step, slot):
    b = step // RC
    r = step % RC
    for j in range(J):
        pltpu.make_async_copy(
            x_hbm.at[b, j, pl.ds(r * SR, SR), :],
            buf.at[slot, j],
            sems.at[slot, j],
        ).start()


def _wait(x_hbm, buf, sems, step, slot):
    b = step // RC
    r = step % RC
    for j in range(J):
        pltpu.make_async_copy(
            x_hbm.at[b, j, pl.ds(r * SR, SR), :],
            buf.at[slot, j],
            sems.at[slot, j],
        ).wait()


def _body(alpha_ref, maskT_ref, x_hbm, o_ref, buf, sems):
    g = pl.program_id(0)
    b = g // RC
    slot = jax.lax.rem(g, NBUF)

    @pl.when(g == 0)
    def _prime():
        for s in range(NBUF):
            _issue(x_hbm, buf, sems, jnp.int32(s), jnp.int32(s))

    @pl.when((g > 0) & (g + NBUF - 1 < NSTEP))
    def _next():
        step = g + NBUF - 1
        _issue(x_hbm, buf, sems, step, jax.lax.rem(step, NBUF))

    _wait(x_hbm, buf, sems, g, slot)

    logits = maskT_ref[:] * alpha_ref[0, 0]
    m = jnp.max(logits, axis=0, keepdims=True)
    e = jnp.exp(logits - m)
    wT = e / jnp.sum(e, axis=0, keepdims=True)
    lane = jax.lax.broadcasted_iota(jnp.int32, (J, B), 1)
    wcol = jnp.sum(jnp.where(lane == b, wT, 0.0), axis=1)

    acc = buf[slot, 0] * wcol[0]
    for j in range(1, J):
        acc += buf[slot, j] * wcol[j]
    o_ref[0] = acc


def kernel(inps, mask, alpha):
    x = inps.reshape(B, J, R, L)
    maskT = mask.T
    alpha2 = jnp.reshape(alpha, (1, 1))
    out = pl.pallas_call(
        _body,
        grid=(NSTEP,),
        in_specs=[
            pl.BlockSpec(memory_space=pltpu.SMEM),
            pl.BlockSpec((J, B), lambda g: (0, 0)),
            pl.BlockSpec(memory_space=pltpu.MemorySpace.HBM),
        ],
        out_specs=pl.BlockSpec((1, SR, L), lambda g: (g // RC, g % RC, 0)),
        out_shape=jax.ShapeDtypeStruct((B, R, L), jnp.float32),
        scratch_shapes=[
            pltpu.VMEM((NBUF, J, SR, L), jnp.float32),
            pltpu.SemaphoreType.DMA((NBUF, J)),
        ],
        compiler_params=pltpu.CompilerParams(
            dimension_semantics=("arbitrary",),
        ),
    )(alpha2, maskT, x)
    sampled = out.reshape(B, 96, 64, 64)
    logp = jnp.zeros((B,), jnp.float32)
    return (sampled, logp)


# speedup vs baseline: 2.5072x; 1.0753x over previous
import jax
import jax.numpy as jnp
from jax.experimental import pallas as pl
from jax.experimental.pallas import tpu as pltpu

B, J = 16, 8
R, L = 96, 64 * 64
SR = 16
RC = R // SR
NSTEP = B * RC
NBUF = 6


def _issue(x_hbm, buf, sems, step, slot):
    b = step // RC
    r = step % RC
    for j in range(J):
        pltpu.make_async_copy(
            x_hbm.at[b, j, pl.ds(r * SR, SR), :],
            buf.at[slot, j],
            sems.at[slot, j],
        ).start()


def _wait(x_hbm, buf, sems, step, slot):
    b = step // RC
    r = step % RC
    for j in range(J):
        pltpu.make_async_copy(
            x_hbm.at[b, j, pl.ds(r * SR, SR), :],
            buf.at[slot, j],
            sems.at[slot, j],
        ).wait()


def _body(alpha_ref, maskT_ref, x_hbm, o_ref, buf, sems):
    g = pl.program_id(0)
    b = g // RC
    slot = jax.lax.rem(g, NBUF)

    @pl.when(g == 0)
    def _prime():
        for s in range(NBUF):
            _issue(x_hbm, buf, sems, jnp.int32(s), jnp.int32(s))

    @pl.when((g > 0) & (g + NBUF - 1 < NSTEP))
    def _next():
        step = g + NBUF - 1
        _issue(x_hbm, buf, sems, step, jax.lax.rem(step, NBUF))

    _wait(x_hbm, buf, sems, g, slot)

    logits = maskT_ref[:] * alpha_ref[0, 0]
    m = jnp.max(logits, axis=0, keepdims=True)
    e = jnp.exp(logits - m)
    wT = e / jnp.sum(e, axis=0, keepdims=True)
    lane = jax.lax.broadcasted_iota(jnp.int32, (J, B), 1)
    wcol = jnp.sum(jnp.where(lane == b, wT, 0.0), axis=1)

    acc = buf[slot, 0] * wcol[0]
    for j in range(1, J):
        acc += buf[slot, j] * wcol[j]
    o_ref[0] = acc


def kernel(inps, mask, alpha):
    x = inps.reshape(B, J, R, L)
    maskT = mask.T
    alpha2 = jnp.reshape(alpha, (1, 1))
    out = pl.pallas_call(
        _body,
        grid=(NSTEP,),
        in_specs=[
            pl.BlockSpec(memory_space=pltpu.SMEM),
            pl.BlockSpec((J, B), lambda g: (0, 0)),
            pl.BlockSpec(memory_space=pltpu.MemorySpace.HBM),
        ],
        out_specs=pl.BlockSpec((1, SR, L), lambda g: (g // RC, g % RC, 0)),
        out_shape=jax.ShapeDtypeStruct((B, R, L), jnp.float32),
        scratch_shapes=[
            pltpu.VMEM((NBUF, J, SR, L), jnp.float32),
            pltpu.SemaphoreType.DMA((NBUF, J)),
        ],
        compiler_params=pltpu.CompilerParams(
            dimension_semantics=("arbitrary",),
        ),
    )(alpha2, maskT, x)
    sampled = out.reshape(B, 96, 64, 64)
    logp = jnp.zeros((B,), jnp.float32)
    return (sampled, logp)
